# baseline (speedup 1.0000x reference)
import jax
import jax.numpy as jnp
from jax.experimental import pallas as pl
from jax.experimental.pallas import tpu as pltpu

K = 16
N = 8192
D = 128
NUM_SEG = 8
R = 256
C = 640
G = N // R

_INF = float("inf")
_FBIG = 3.0e38


def _extract_topk(v, idx):
    vs, js = [], []
    for _ in range(K):
        m = jnp.min(v, axis=1, keepdims=True)
        eq = v == m
        sel = jnp.min(jnp.where(eq, idx, _FBIG), axis=1, keepdims=True)
        vs.append(m)
        js.append(sel)
        v = jnp.where(eq, _INF, v)
    return jnp.concatenate(vs, axis=1), jnp.concatenate(js, axis=1)


def _knn_block(tlo_ref, tcnt_ref, rows_ref, lo_ref, hi_ref, xfull_ref, out_ref):
    i = pl.program_id(0)
    rows = rows_ref[...]
    sqr = jnp.sum(rows * rows, axis=1)
    row_ids = i * R + jax.lax.broadcasted_iota(jnp.int32, (R, 1), 0)
    lo = lo_ref[0, 0, :].reshape(R, 1)
    hi = hi_ref[0, 0, :].reshape(R, 1)

    tlo = tlo_ref[i]
    tcnt = tcnt_ref[i]

    def body(t, carry):
        bv, bi = carry
        c0 = tlo + t * C
        cols = xfull_ref[pl.ds(c0, C), :]
        sqc = jnp.sum(cols * cols, axis=1)
        dot = jax.lax.dot_general(
            rows, cols, (((1,), (1,)), ((), ())),
            preferred_element_type=jnp.float32)
        d2 = sqr[:, None] + sqc[None, :] - 2.0 * dot
        col_ids = c0 + jax.lax.broadcasted_iota(jnp.int32, (1, C), 1)
        valid = (col_ids >= lo) & (col_ids < hi) & (col_ids != row_ids)
        d2 = jnp.where(valid, d2, _INF)
        colf = col_ids.astype(jnp.float32)

        cand_v = jnp.concatenate([bv, d2], axis=1)
        cand_i = jnp.concatenate(
            [bi, jnp.broadcast_to(colf, (R, C))], axis=1)
        return _extract_topk(cand_v, cand_i)

    init = (jnp.full((R, K), _INF, jnp.float32),
            jnp.full((R, K), _FBIG, jnp.float32))
    _, bi = jax.lax.fori_loop(0, tcnt, body, init)
    out_ref[...] = bi.astype(jnp.int32)


def _compiler_params():
    params = getattr(pltpu, "CompilerParams", None)
    if params is None:
        params = pltpu.TPUCompilerParams
    return params(dimension_semantics=("parallel",))


def _knn_indices(x, tlo, tcnt, lo3, hi3):
    grid_spec = pltpu.PrefetchScalarGridSpec(
        num_scalar_prefetch=2,
        grid=(G,),
        in_specs=[
            pl.BlockSpec((R, D), lambda i, *_: (i, 0)),
            pl.BlockSpec((1, 1, R), lambda i, *_: (i, 0, 0)),
            pl.BlockSpec((1, 1, R), lambda i, *_: (i, 0, 0)),
            pl.BlockSpec((N, D), lambda i, *_: (0, 0)),
        ],
        out_specs=pl.BlockSpec((R, K), lambda i, *_: (i, 0)),
    )
    return pl.pallas_call(
        _knn_block,
        grid_spec=grid_spec,
        out_shape=jax.ShapeDtypeStruct((N, K), jnp.int32),
        compiler_params=_compiler_params(),
    )(tlo, tcnt, x, lo3, hi3, x)


def kernel(x, batch):
    b32 = batch.astype(jnp.int32)
    bounds = jnp.searchsorted(
        b32, jnp.arange(NUM_SEG + 1, dtype=jnp.int32)).astype(jnp.int32)
    lo = bounds[b32]
    hi = bounds[b32 + 1]
    blk_lo = lo[::R]
    blk_hi = hi[R - 1::R]
    tstart = (blk_lo // 8) * 8
    tcnt = (blk_hi - tstart + C - 1) // C
    tlo = jnp.minimum(tstart, N - tcnt * C)
    idx = _knn_indices(x, tlo, tcnt, lo.reshape(G, 1, R), hi.reshape(G, 1, R))
    row = jnp.repeat(jnp.arange(N, dtype=jnp.int32), K)
    return jnp.stack([row, idx.reshape(-1)], axis=0)


# speedup vs baseline: 1.1228x; 1.1228x over previous
import jax
import jax.numpy as jnp
from jax.experimental import pallas as pl
from jax.experimental.pallas import tpu as pltpu

K = 16
N = 8192
D = 128
NUM_SEG = 8
R = 256
C = 576
G = N // R

_INF = float("inf")
_FBIG = 3.0e38


def _extract_topk(v, idx):
    vs, js = [], []
    for _ in range(K):
        m = jnp.min(v, axis=1, keepdims=True)
        eq = v == m
        sel = jnp.min(jnp.where(eq, idx, _FBIG), axis=1, keepdims=True)
        vs.append(m)
        js.append(sel)
        v = jnp.where(eq, _INF, v)
    return jnp.concatenate(vs, axis=1), jnp.concatenate(js, axis=1)


def _knn_block(tlo_ref, tcnt_ref, rows_ref, lo_ref, hi_ref, xfull_ref, out_ref):
    i = pl.program_id(0)
    rows = rows_ref[...]
    sqr = jnp.sum(rows * rows, axis=1)
    row_ids = i * R + jax.lax.broadcasted_iota(jnp.int32, (R, 1), 0)
    lo = lo_ref[0, 0, :].reshape(R, 1)
    hi = hi_ref[0, 0, :].reshape(R, 1)

    tlo = tlo_ref[i]
    tcnt = tcnt_ref[i]

    def body(t, carry):
        bv, bi = carry
        c0 = tlo + t * C
        cols = xfull_ref[pl.ds(c0, C), :]
        sqc = jnp.sum(cols * cols, axis=1)
        dot = jax.lax.dot_general(
            rows, cols, (((1,), (1,)), ((), ())),
            preferred_element_type=jnp.float32)
        d2 = sqr[:, None] + sqc[None, :] - 2.0 * dot
        col_ids = c0 + jax.lax.broadcasted_iota(jnp.int32, (1, C), 1)
        valid = (col_ids >= lo) & (col_ids < hi) & (col_ids != row_ids)
        d2 = jnp.where(valid, d2, _INF)
        colf = col_ids.astype(jnp.float32)

        cand_v = jnp.concatenate([bv, d2], axis=1)
        cand_i = jnp.concatenate(
            [bi, jnp.broadcast_to(colf, (R, C))], axis=1)
        return _extract_topk(cand_v, cand_i)

    init = (jnp.full((R, K), _INF, jnp.float32),
            jnp.full((R, K), _FBIG, jnp.float32))
    _, bi = jax.lax.fori_loop(0, tcnt, body, init)
    out_ref[...] = bi.astype(jnp.int32)


def _compiler_params():
    params = getattr(pltpu, "CompilerParams", None)
    if params is None:
        params = pltpu.TPUCompilerParams
    return params(dimension_semantics=("parallel",))


def _knn_indices(x, tlo, tcnt, lo3, hi3):
    grid_spec = pltpu.PrefetchScalarGridSpec(
        num_scalar_prefetch=2,
        grid=(G,),
        in_specs=[
            pl.BlockSpec((R, D), lambda i, *_: (i, 0)),
            pl.BlockSpec((1, 1, R), lambda i, *_: (i, 0, 0)),
            pl.BlockSpec((1, 1, R), lambda i, *_: (i, 0, 0)),
            pl.BlockSpec((N, D), lambda i, *_: (0, 0)),
        ],
        out_specs=pl.BlockSpec((R, K), lambda i, *_: (i, 0)),
    )
    return pl.pallas_call(
        _knn_block,
        grid_spec=grid_spec,
        out_shape=jax.ShapeDtypeStruct((N, K), jnp.int32),
        compiler_params=_compiler_params(),
    )(tlo, tcnt, x, lo3, hi3, x)


def kernel(x, batch):
    b32 = batch.astype(jnp.int32)
    bounds = jnp.searchsorted(
        b32, jnp.arange(NUM_SEG + 1, dtype=jnp.int32)).astype(jnp.int32)
    lo = bounds[b32]
    hi = bounds[b32 + 1]
    blk_lo = lo[::R]
    blk_hi = hi[R - 1::R]
    tstart = (blk_lo // 8) * 8
    tcnt = (blk_hi - tstart + C - 1) // C
    tlo = jnp.minimum(tstart, N - tcnt * C)
    idx = _knn_indices(x, tlo, tcnt, lo.reshape(G, 1, R), hi.reshape(G, 1, R))
    row = jnp.repeat(jnp.arange(N, dtype=jnp.int32), K)
    return jnp.stack([row, idx.reshape(-1)], axis=0)


# speedup vs baseline: 1.2412x; 1.1055x over previous
import jax
import jax.numpy as jnp
from jax.experimental import pallas as pl
from jax.experimental.pallas import tpu as pltpu

K = 16
N = 8192
D = 128
NUM_SEG = 8
R = 256
C = 1088
G = N // R

_INF = float("inf")
_FBIG = 3.0e38


def _extract_topk(v, idx):
    vs, js = [], []
    for _ in range(K):
        m = jnp.min(v, axis=1, keepdims=True)
        eq = v == m
        sel = jnp.min(jnp.where(eq, idx, _FBIG), axis=1, keepdims=True)
        vs.append(m)
        js.append(sel)
        v = jnp.where(eq, _INF, v)
    return jnp.concatenate(vs, axis=1), jnp.concatenate(js, axis=1)


def _knn_block(tlo_ref, tcnt_ref, rows_ref, lo_ref, hi_ref, xfull_ref, out_ref):
    i = pl.program_id(0)
    rows = rows_ref[...]
    sqr = jnp.sum(rows * rows, axis=1)
    row_ids = i * R + jax.lax.broadcasted_iota(jnp.int32, (R, 1), 0)
    lo = lo_ref[0, 0, :].reshape(R, 1)
    hi = hi_ref[0, 0, :].reshape(R, 1)

    tlo = tlo_ref[i]
    tcnt = tcnt_ref[i]

    def body(t, carry):
        bv, bi = carry
        c0 = tlo + t * C
        cols = xfull_ref[pl.ds(c0, C), :]
        sqc = jnp.sum(cols * cols, axis=1)
        dot = jax.lax.dot_general(
            rows, cols, (((1,), (1,)), ((), ())),
            preferred_element_type=jnp.float32)
        d2 = sqr[:, None] + sqc[None, :] - 2.0 * dot
        col_ids = c0 + jax.lax.broadcasted_iota(jnp.int32, (1, C), 1)
        valid = (col_ids >= lo) & (col_ids < hi) & (col_ids != row_ids)
        d2 = jnp.where(valid, d2, _INF)
        colf = col_ids.astype(jnp.float32)

        cand_v = jnp.concatenate([bv, d2], axis=1)
        cand_i = jnp.concatenate(
            [bi, jnp.broadcast_to(colf, (R, C))], axis=1)
        return _extract_topk(cand_v, cand_i)

    init = (jnp.full((R, K), _INF, jnp.float32),
            jnp.full((R, K), _FBIG, jnp.float32))
    _, bi = jax.lax.fori_loop(0, tcnt, body, init)
    out_ref[...] = bi.astype(jnp.int32)


def _compiler_params():
    params = getattr(pltpu, "CompilerParams", None)
    if params is None:
        params = pltpu.TPUCompilerParams
    return params(dimension_semantics=("parallel",))


def _knn_indices(x, tlo, tcnt, lo3, hi3):
    grid_spec = pltpu.PrefetchScalarGridSpec(
        num_scalar_prefetch=2,
        grid=(G,),
        in_specs=[
            pl.BlockSpec((R, D), lambda i, *_: (i, 0)),
            pl.BlockSpec((1, 1, R), lambda i, *_: (i, 0, 0)),
            pl.BlockSpec((1, 1, R), lambda i, *_: (i, 0, 0)),
            pl.BlockSpec((N, D), lambda i, *_: (0, 0)),
        ],
        out_specs=pl.BlockSpec((R, K), lambda i, *_: (i, 0)),
    )
    return pl.pallas_call(
        _knn_block,
        grid_spec=grid_spec,
        out_shape=jax.ShapeDtypeStruct((N, K), jnp.int32),
        compiler_params=_compiler_params(),
    )(tlo, tcnt, x, lo3, hi3, x)


def kernel(x, batch):
    b32 = batch.astype(jnp.int32)
    bounds = jnp.searchsorted(
        b32, jnp.arange(NUM_SEG + 1, dtype=jnp.int32)).astype(jnp.int32)
    lo = bounds[b32]
    hi = bounds[b32 + 1]
    blk_lo = lo[::R]
    blk_hi = hi[R - 1::R]
    tstart = (blk_lo // 8) * 8
    tcnt = (blk_hi - tstart + C - 1) // C
    tlo = jnp.minimum(tstart, N - tcnt * C)
    idx = _knn_indices(x, tlo, tcnt, lo.reshape(G, 1, R), hi.reshape(G, 1, R))
    row = jnp.repeat(jnp.arange(N, dtype=jnp.int32), K)
    return jnp.stack([row, idx.reshape(-1)], axis=0)
